# baseline (speedup 1.0000x reference)
import jax
import jax.numpy as jnp
import numpy as np
from jax import lax
from jax.experimental import pallas as pl
from jax.experimental.pallas import tpu as pltpu
from jax.experimental.pallas import tpu_sc as plsc

B = 16384
C = 100
MAX_POS = 32
MAX_NEG = 64

NUM_CORES = 2
NUM_SUBCORES = 16
NUM_WORKERS = NUM_CORES * NUM_SUBCORES
LANES = 16

FAST = 512
FAST_CHUNKS = FAST // LANES
SUB = 128
SUB_CHUNKS = SUB // LANES
NUM_SUBS = B // SUB

_U32 = np.uint32


def _rol(x, r):
    r = _U32(r)
    return (x << r) | (x >> _U32(32 - r))


def _threefry2x32(k1, k2, x1, x2):
    ks0 = _U32(k1) * np.ones_like(x1)
    ks1 = _U32(k2) * np.ones_like(x1)
    ks2 = ks0 ^ ks1 ^ _U32(0x1BD11BDA)
    rot1 = (13, 15, 26, 6)
    rot2 = (17, 29, 16, 24)
    a = x1 + ks0
    b = x2 + ks1

    def rounds(a, b, rots):
        for r in rots:
            a = a + b
            b = _rol(b, r)
            b = a ^ b
        return a, b

    a, b = rounds(a, b, rot1); a = a + ks1; b = b + ks2 + _U32(1)
    a, b = rounds(a, b, rot2); a = a + ks2; b = b + ks0 + _U32(2)
    a, b = rounds(a, b, rot1); a = a + ks0; b = b + ks1 + _U32(3)
    a, b = rounds(a, b, rot2); a = a + ks1; b = b + ks2 + _U32(4)
    a, b = rounds(a, b, rot1); a = a + ks2; b = b + ks0 + _U32(5)
    return a, b


def _uniform_01(key, n):
    a, b = _threefry2x32(key[0], key[1], np.zeros(n, _U32),
                         np.arange(n, dtype=_U32))
    bits = a ^ b
    fb = (bits >> _U32(9)) | _U32(0x3F800000)
    return fb.view(np.float32) - np.float32(1.0)


def _sampling_perms():
    pp = np.empty((C, B), np.int32)
    pn = np.empty((C, B), np.int32)
    for c in range(C):
        a, b = _threefry2x32(0, 42, np.array([0], _U32), np.array([c], _U32))
        kc = (a[0], b[0])
        a, b = _threefry2x32(kc[0], kc[1], np.array([0, 0], _U32),
                             np.array([0, 1], _U32))
        kp, kn = (a[0], b[0]), (a[1], b[1])
        pp[c] = np.argsort(_uniform_01(kp, B), kind="stable")
        pn[c] = np.argsort(_uniform_01(kn, B), kind="stable")
    return (pp.reshape(C, B // LANES, LANES),
            pn.reshape(C, B // LANES, LANES))


_PP, _PN = _sampling_perms()

def _chunk_scan(tvals, base_row, idx, c, cj, want, tv, invert, out_idx):
    fi = idx * C + c
    lane = jnp.bitwise_and(fi, 15)
    row = lax.iota(jnp.int32, LANES) + base_row
    t = plsc.load_gather(tvals, [row, lane])
    m = (t != tv) if invert else (t == tv)
    mi = m.astype(jnp.int32)
    cs = plsc.cumsum(mi)
    dst = cj + cs - 1
    keep = jnp.logical_and(m, dst < want)
    dst_safe = jnp.where(keep, dst, 0)
    plsc.store_scatter(out_idx, [dst_safe], idx, mask=keep)
    return cj + jnp.sum(mi)


def _fill_fidx(win, fidx, c, nchunks):
    for j in range(nchunks):
        idx = win[j]
        fi = idx * C + c
        row = lax.shift_right_logical(fi, 4)
        fidx[pl.ds(j * LANES, LANES)] = row


def _cont_scan(perm_hbm, tflat, c, out_idx, want, tv, invert, start_cnt,
               start_sub, cwin, cfidx, ctvals, sem):
    def sub_work(w, cnt):
        pltpu.sync_copy(perm_hbm.at[c, pl.ds(w * SUB_CHUNKS, SUB_CHUNKS)],
                        cwin)
        _fill_fidx(cwin, cfidx, c, SUB_CHUNKS)
        pltpu.async_copy(tflat.at[cfidx], ctvals, sem).wait()
        for j in range(SUB_CHUNKS):
            cnt = _chunk_scan(ctvals, j * LANES, cwin[j], c, cnt, want, tv,
                              invert, out_idx)
        return cnt

    def sub_body(w, cnt):
        return lax.cond(cnt < want, lambda x: sub_work(w, x), lambda x: x,
                        cnt)

    return lax.fori_loop(start_sub, NUM_SUBS, sub_body, start_cnt)


def _sc_body(tflat, lflat, pp, pn, pos_out, neg_out,
             pwin, nwin, pfidx, nfidx, ptvals, ntvals,
             cwin, cfidx, ctvals,
             pidx, nidx, vfidx, lvals, pvals, nvals,
             semw, semg, semc, semv):
    wid = lax.axis_index("s") * NUM_CORES + lax.axis_index("c")

    def do_class(c):
        cp_p = pltpu.async_copy(pp.at[c, pl.ds(0, FAST_CHUNKS)], pwin, semw)
        cp_n = pltpu.async_copy(pn.at[c, pl.ds(0, FAST_CHUNKS)], nwin, semw)
        cp_p.wait()
        cp_n.wait()

        _fill_fidx(pwin, pfidx, c, FAST_CHUNKS)
        _fill_fidx(nwin, nfidx, c, FAST_CHUNKS)
        handles = []
        for q in range(FAST // SUB):
            handles.append(pltpu.async_copy(
                tflat.at[pfidx.at[pl.ds(q * SUB, SUB)]],
                ptvals.at[pl.ds(q * SUB, SUB)], semg))
        for q in range(FAST // SUB):
            handles.append(pltpu.async_copy(
                tflat.at[nfidx.at[pl.ds(q * SUB, SUB)]],
                ntvals.at[pl.ds(q * SUB, SUB)], semg))
        for h in handles:
            h.wait()

        def fast_scan(win, tvals, out_idx, want, tv):
            def chunk_body(j, cj):
                def work(x):
                    return _chunk_scan(tvals, j * LANES, win[j], c, x, want,
                                       tv, False, out_idx)
                return lax.cond(cj < want, work, lambda x: x, cj)
            return lax.fori_loop(0, FAST_CHUNKS, chunk_body, jnp.int32(0))

        npos = fast_scan(pwin, ptvals, pidx, MAX_POS, 1)
        nneg = fast_scan(nwin, ntvals, nidx, MAX_NEG, 0)

        npos = lax.cond(
            npos < MAX_POS,
            lambda x: _cont_scan(pp, tflat, c, pidx, MAX_POS, 1, False, x,
                                 FAST // SUB, cwin, cfidx, ctvals, semc),
            lambda x: x, npos)
        nneg = lax.cond(
            nneg < MAX_NEG,
            lambda x: _cont_scan(pn, tflat, c, nidx, MAX_NEG, 0, False, x,
                                 FAST // SUB, cwin, cfidx, ctvals, semc),
            lambda x: x, nneg)

        @pl.when(npos < MAX_POS)
        def _():
            _cont_scan(pp, tflat, c, pidx, MAX_POS, 1, True, npos, 0,
                       cwin, cfidx, ctvals, semc)

        @pl.when(nneg < MAX_NEG)
        def _():
            _cont_scan(pn, tflat, c, nidx, MAX_NEG, 0, True, nneg, 0,
                       cwin, cfidx, ctvals, semc)

        for s in range(MAX_POS // LANES):
            sel = pidx[pl.ds(s * LANES, LANES)]
            fi = sel * C + c
            vfidx[pl.ds(s * LANES, LANES)] = lax.shift_right_logical(fi, 4)
        for s in range(MAX_NEG // LANES):
            sel = nidx[pl.ds(s * LANES, LANES)]
            fi = sel * C + c
            vfidx[pl.ds(MAX_POS + s * LANES, LANES)] = (
                lax.shift_right_logical(fi, 4))
        pltpu.async_copy(lflat.at[vfidx], lvals, semv).wait()

        for s in range(MAX_POS // LANES):
            sel = pidx[pl.ds(s * LANES, LANES)]
            lane = jnp.bitwise_and(sel * C + c, 15)
            row = lax.iota(jnp.int32, LANES) + s * LANES
            pvals[pl.ds(s * LANES, LANES)] = plsc.load_gather(
                lvals, [row, lane])
        for s in range(MAX_NEG // LANES):
            sel = nidx[pl.ds(s * LANES, LANES)]
            lane = jnp.bitwise_and(sel * C + c, 15)
            row = lax.iota(jnp.int32, LANES) + MAX_POS + s * LANES
            nvals[pl.ds(s * LANES, LANES)] = plsc.load_gather(
                lvals, [row, lane])
        pltpu.sync_copy(pvals, pos_out.at[c])
        pltpu.sync_copy(nvals, neg_out.at[c])

    def k_body(k, carry):
        c = wid + NUM_WORKERS * k

        @pl.when(c < C)
        def _():
            do_class(c)

        return carry

    lax.fori_loop(0, (C + NUM_WORKERS - 1) // NUM_WORKERS, k_body,
                  jnp.int32(0))


def _make_sc_sampler():
    mesh = plsc.VectorSubcoreMesh(core_axis_name="c", subcore_axis_name="s",
                                  num_cores=NUM_CORES,
                                  num_subcores=NUM_SUBCORES)
    return pl.kernel(
        _sc_body,
        out_type=[
            jax.ShapeDtypeStruct((C, MAX_POS), jnp.float32),
            jax.ShapeDtypeStruct((C, MAX_NEG), jnp.float32),
        ],
        mesh=mesh,
        compiler_params=pltpu.CompilerParams(needs_layout_passes=False,
                                             use_tc_tiling_on_sc=False),
        scratch_types=[
            pltpu.VMEM((FAST_CHUNKS, LANES), jnp.int32),
            pltpu.VMEM((FAST_CHUNKS, LANES), jnp.int32),
            pltpu.VMEM((FAST,), jnp.int32),
            pltpu.VMEM((FAST,), jnp.int32),
            pltpu.VMEM((FAST, LANES), jnp.int32),
            pltpu.VMEM((FAST, LANES), jnp.int32),
            pltpu.VMEM((SUB_CHUNKS, LANES), jnp.int32),
            pltpu.VMEM((SUB,), jnp.int32),
            pltpu.VMEM((SUB, LANES), jnp.int32),
            pltpu.VMEM((MAX_POS,), jnp.int32),
            pltpu.VMEM((MAX_NEG,), jnp.int32),
            pltpu.VMEM((MAX_POS + MAX_NEG,), jnp.int32),
            pltpu.VMEM((MAX_POS + MAX_NEG, LANES), jnp.float32),
            pltpu.VMEM((MAX_POS,), jnp.float32),
            pltpu.VMEM((MAX_NEG,), jnp.float32),
            pltpu.SemaphoreType.DMA,
            pltpu.SemaphoreType.DMA,
            pltpu.SemaphoreType.DMA,
            pltpu.SemaphoreType.DMA,
        ],
    )


def _loss_body(p_ref, n_ref, o_ref):
    n = n_ref[...]
    total = jnp.float32(0.0)
    for i in range(MAX_POS):
        d = n - p_ref[:, i][:, None]
        total = total + jnp.sum(jnp.logaddexp(d, 0.0))
    o_ref[0, 0] = total / jnp.float32(C * MAX_POS * MAX_NEG)


def _tc_loss(pos_vals, neg_vals):
    return pl.pallas_call(
        _loss_body,
        out_shape=jax.ShapeDtypeStruct((1, 1), jnp.float32),
        out_specs=pl.BlockSpec(memory_space=pltpu.SMEM),
    )(pos_vals, neg_vals)


def kernel(logits, targets):
    tflat = targets.astype(jnp.int32).reshape(B * C // LANES, LANES)
    lflat = logits.reshape(B * C // LANES, LANES)
    sampler = _make_sc_sampler()
    pos_vals, neg_vals = sampler(tflat, lflat, _PP, _PN)
    loss = _tc_loss(pos_vals, neg_vals)
    return jnp.reshape(loss, ())


# speedup vs baseline: 1.0316x; 1.0316x over previous
import jax
import jax.numpy as jnp
import numpy as np
from jax import lax
from jax.experimental import pallas as pl
from jax.experimental.pallas import tpu as pltpu
from jax.experimental.pallas import tpu_sc as plsc

B = 16384
C = 100
MAX_POS = 32
MAX_NEG = 64

NUM_CORES = 2
NUM_SUBCORES = 16
NUM_WORKERS = NUM_CORES * NUM_SUBCORES
LANES = 16

FAST = 512
FAST_CHUNKS = FAST // LANES
SUB = 128
SUB_CHUNKS = SUB // LANES
NUM_SUBS = B // SUB
MWORDS = B // 32

_U32 = np.uint32


def _rol(x, r):
    r = _U32(r)
    return (x << r) | (x >> _U32(32 - r))


def _threefry2x32(k1, k2, x1, x2):
    ks0 = _U32(k1) * np.ones_like(x1)
    ks1 = _U32(k2) * np.ones_like(x1)
    ks2 = ks0 ^ ks1 ^ _U32(0x1BD11BDA)
    rot1 = (13, 15, 26, 6)
    rot2 = (17, 29, 16, 24)
    a = x1 + ks0
    b = x2 + ks1

    def rounds(a, b, rots):
        for r in rots:
            a = a + b
            b = _rol(b, r)
            b = a ^ b
        return a, b

    a, b = rounds(a, b, rot1); a = a + ks1; b = b + ks2 + _U32(1)
    a, b = rounds(a, b, rot2); a = a + ks2; b = b + ks0 + _U32(2)
    a, b = rounds(a, b, rot1); a = a + ks0; b = b + ks1 + _U32(3)
    a, b = rounds(a, b, rot2); a = a + ks1; b = b + ks2 + _U32(4)
    a, b = rounds(a, b, rot1); a = a + ks2; b = b + ks0 + _U32(5)
    return a, b


def _uniform_01(key, n):
    a, b = _threefry2x32(key[0], key[1], np.zeros(n, _U32),
                         np.arange(n, dtype=_U32))
    bits = a ^ b
    fb = (bits >> _U32(9)) | _U32(0x3F800000)
    return fb.view(np.float32) - np.float32(1.0)


def _sampling_perms():
    pp = np.empty((C, B), np.int32)
    pn = np.empty((C, B), np.int32)
    for c in range(C):
        a, b = _threefry2x32(0, 42, np.array([0], _U32), np.array([c], _U32))
        kc = (a[0], b[0])
        a, b = _threefry2x32(kc[0], kc[1], np.array([0, 0], _U32),
                             np.array([0, 1], _U32))
        kp, kn = (a[0], b[0]), (a[1], b[1])
        pp[c] = np.argsort(_uniform_01(kp, B), kind="stable")
        pn[c] = np.argsort(_uniform_01(kn, B), kind="stable")
    return (pp.reshape(C, B // LANES, LANES),
            pn.reshape(C, B // LANES, LANES))


_PP, _PN = _sampling_perms()


_MROWS = 4096
_MGRID = B // _MROWS


def _mask_body(t_ref, pm_ref, nm_ref):
    t = t_ref[...]
    tt = t.reshape(_MROWS // 32, 32, C)
    sh = lax.broadcasted_iota(jnp.int32, (1, 32, 1), 1)
    one = jnp.int32(1)
    zero = jnp.int32(0)
    pm = jnp.sum(jnp.where(tt == 1, one, zero) << sh, axis=1,
                 dtype=jnp.int32)
    nm = jnp.sum(jnp.where(tt == 0, one, zero) << sh, axis=1,
                 dtype=jnp.int32)
    pm_ref[...] = pm.T
    nm_ref[...] = nm.T


def _tc_masks(targets):
    return pl.pallas_call(
        _mask_body,
        grid=(_MGRID,),
        in_specs=[pl.BlockSpec((_MROWS, C), lambda i: (i, 0))],
        out_specs=[pl.BlockSpec((C, _MROWS // 32), lambda i: (0, i)),
                   pl.BlockSpec((C, _MROWS // 32), lambda i: (0, i))],
        out_shape=[jax.ShapeDtypeStruct((C, MWORDS), jnp.int32),
                   jax.ShapeDtypeStruct((C, MWORDS), jnp.int32)],
    )(targets)


def _chunk_scan(mrow, base, idx, cj, want, invert, out_idx):
    word = plsc.load_gather(mrow, [lax.shift_right_logical(idx, 5) + base])
    bit = jnp.bitwise_and(
        lax.shift_right_logical(word, jnp.bitwise_and(idx, 31)), 1)
    m = (bit == 0) if invert else (bit == 1)
    mi = m.astype(jnp.int32)
    cs = plsc.cumsum(mi)
    dst = cj + cs - 1
    keep = jnp.logical_and(m, dst < want)
    dst_safe = jnp.where(keep, dst, 0)
    plsc.store_scatter(out_idx, [dst_safe], idx, mask=keep)
    return cj + jnp.sum(mi)


def _cont_scan(perm_hbm, c, mrow, base, out_idx, want, invert, start_cnt,
               start_sub, cwin, sem):
    def sub_work(w, cnt):
        pltpu.async_copy(perm_hbm.at[c, pl.ds(w * SUB_CHUNKS, SUB_CHUNKS)],
                         cwin, sem).wait()
        for j in range(SUB_CHUNKS):
            cnt = _chunk_scan(mrow, base, cwin[j], cnt, want, invert,
                              out_idx)
        return cnt

    def sub_body(w, cnt):
        return lax.cond(cnt < want, lambda x: sub_work(w, x), lambda x: x,
                        cnt)

    return lax.fori_loop(start_sub, NUM_SUBS, sub_body, start_cnt)


_KCLASSES = (C + NUM_WORKERS - 1) // NUM_WORKERS


def _sc_body(lflat, pmT, nmT, pp, pn, pos_out, neg_out,
             pwin, nwin, pmrow, nmrow, cwin,
             pidx, nidx, vfidx, lvals, pvals, nvals,
             semw, semc, semv, semo):
    wid = lax.axis_index("s") * NUM_CORES + lax.axis_index("c")

    stage = []
    for k in range(_KCLASSES):
        cm = jnp.minimum(wid + NUM_WORKERS * k, C - 1)
        stage.append([
            pltpu.async_copy(pmT.at[cm], pmrow.at[pl.ds(k * MWORDS, MWORDS)],
                             semw),
            pltpu.async_copy(nmT.at[cm], nmrow.at[pl.ds(k * MWORDS, MWORDS)],
                             semw),
            pltpu.async_copy(pp.at[cm, pl.ds(0, FAST_CHUNKS)],
                             pwin.at[pl.ds(k * FAST_CHUNKS, FAST_CHUNKS)],
                             semw),
            pltpu.async_copy(pn.at[cm, pl.ds(0, FAST_CHUNKS)],
                             nwin.at[pl.ds(k * FAST_CHUNKS, FAST_CHUNKS)],
                             semw),
        ])

    def do_class(k, c):
        base = k * MWORDS

        def fast_scan(win, mrow, out_idx, want):
            def chunk_body(j, cj):
                def work(x):
                    return _chunk_scan(mrow, base, win[k * FAST_CHUNKS + j],
                                       x, want, False, out_idx)
                return lax.cond(cj < want, work, lambda x: x, cj)
            return lax.fori_loop(0, FAST_CHUNKS, chunk_body, jnp.int32(0))

        npos = fast_scan(pwin, pmrow, pidx, MAX_POS)
        nneg = fast_scan(nwin, nmrow, nidx, MAX_NEG)

        npos = lax.cond(
            npos < MAX_POS,
            lambda x: _cont_scan(pp, c, pmrow, base, pidx, MAX_POS, False,
                                 x, FAST // SUB, cwin, semc),
            lambda x: x, npos)
        nneg = lax.cond(
            nneg < MAX_NEG,
            lambda x: _cont_scan(pn, c, nmrow, base, nidx, MAX_NEG, False,
                                 x, FAST // SUB, cwin, semc),
            lambda x: x, nneg)

        @pl.when(npos < MAX_POS)
        def _():
            _cont_scan(pp, c, pmrow, base, pidx, MAX_POS, True, npos, 0,
                       cwin, semc)

        @pl.when(nneg < MAX_NEG)
        def _():
            _cont_scan(pn, c, nmrow, base, nidx, MAX_NEG, True, nneg, 0,
                       cwin, semc)

        for s in range(MAX_POS // LANES):
            sel = pidx[pl.ds(s * LANES, LANES)]
            fi = sel * C + c
            vfidx[pl.ds(s * LANES, LANES)] = lax.shift_right_logical(fi, 4)
        for s in range(MAX_NEG // LANES):
            sel = nidx[pl.ds(s * LANES, LANES)]
            fi = sel * C + c
            vfidx[pl.ds(MAX_POS + s * LANES, LANES)] = (
                lax.shift_right_logical(fi, 4))
        pltpu.async_copy(lflat.at[vfidx], lvals, semv).wait()

        for s in range(MAX_POS // LANES):
            sel = pidx[pl.ds(s * LANES, LANES)]
            lane = jnp.bitwise_and(sel * C + c, 15)
            row = lax.iota(jnp.int32, LANES) + s * LANES
            pvals[pl.ds(s * LANES, LANES)] = plsc.load_gather(
                lvals, [row, lane])
        for s in range(MAX_NEG // LANES):
            sel = nidx[pl.ds(s * LANES, LANES)]
            lane = jnp.bitwise_and(sel * C + c, 15)
            row = lax.iota(jnp.int32, LANES) + MAX_POS + s * LANES
            nvals[pl.ds(s * LANES, LANES)] = plsc.load_gather(
                lvals, [row, lane])
        h1 = pltpu.async_copy(pvals, pos_out.at[c], semo)
        h2 = pltpu.async_copy(nvals, neg_out.at[c], semo)
        h1.wait()
        h2.wait()

    for k in range(_KCLASSES):
        c = wid + NUM_WORKERS * k
        for h in stage[k]:
            h.wait()

        @pl.when(c < C)
        def _(k=k, c=c):
            do_class(k, c)


def _make_sc_sampler():
    mesh = plsc.VectorSubcoreMesh(core_axis_name="c", subcore_axis_name="s",
                                  num_cores=NUM_CORES,
                                  num_subcores=NUM_SUBCORES)
    return pl.kernel(
        _sc_body,
        out_type=[
            jax.ShapeDtypeStruct((C, MAX_POS), jnp.float32),
            jax.ShapeDtypeStruct((C, MAX_NEG), jnp.float32),
        ],
        mesh=mesh,
        compiler_params=pltpu.CompilerParams(needs_layout_passes=False,
                                             use_tc_tiling_on_sc=False),
        scratch_types=[
            pltpu.VMEM((_KCLASSES * FAST_CHUNKS, LANES), jnp.int32),
            pltpu.VMEM((_KCLASSES * FAST_CHUNKS, LANES), jnp.int32),
            pltpu.VMEM((_KCLASSES * MWORDS,), jnp.int32),
            pltpu.VMEM((_KCLASSES * MWORDS,), jnp.int32),
            pltpu.VMEM((SUB_CHUNKS, LANES), jnp.int32),
            pltpu.VMEM((MAX_POS,), jnp.int32),
            pltpu.VMEM((MAX_NEG,), jnp.int32),
            pltpu.VMEM((MAX_POS + MAX_NEG,), jnp.int32),
            pltpu.VMEM((MAX_POS + MAX_NEG, LANES), jnp.float32),
            pltpu.VMEM((MAX_POS,), jnp.float32),
            pltpu.VMEM((MAX_NEG,), jnp.float32),
            pltpu.SemaphoreType.DMA,
            pltpu.SemaphoreType.DMA,
            pltpu.SemaphoreType.DMA,
            pltpu.SemaphoreType.DMA,
        ],
    )


def _loss_body(p_ref, n_ref, o_ref):
    n = n_ref[...]
    total = jnp.float32(0.0)
    for i in range(MAX_POS):
        d = n - p_ref[:, i][:, None]
        total = total + jnp.sum(jnp.logaddexp(d, 0.0))
    o_ref[0, 0] = total / jnp.float32(C * MAX_POS * MAX_NEG)


def _tc_loss(pos_vals, neg_vals):
    return pl.pallas_call(
        _loss_body,
        out_shape=jax.ShapeDtypeStruct((1, 1), jnp.float32),
        out_specs=pl.BlockSpec(memory_space=pltpu.SMEM),
    )(pos_vals, neg_vals)


def kernel(logits, targets):
    lflat = logits.reshape(B * C // LANES, LANES)
    pmT, nmT = _tc_masks(targets.astype(jnp.int32))
    sampler = _make_sc_sampler()
    pos_vals, neg_vals = sampler(lflat, pmT, nmT, _PP, _PN)
    loss = _tc_loss(pos_vals, neg_vals)
    return jnp.reshape(loss, ())


# speedup vs baseline: 1.1870x; 1.1507x over previous
import jax
import jax.numpy as jnp
import numpy as np
from jax import lax
from jax.experimental import pallas as pl
from jax.experimental.pallas import tpu as pltpu
from jax.experimental.pallas import tpu_sc as plsc

B = 16384
C = 100
MAX_POS = 32
MAX_NEG = 64

NUM_CORES = 2
NUM_SUBCORES = 16
NUM_WORKERS = NUM_CORES * NUM_SUBCORES
LANES = 16

FAST = 512
FAST_CHUNKS = FAST // LANES
SUB = 128
SUB_CHUNKS = SUB // LANES
NUM_SUBS = B // SUB
MWORDS = B // 32

_U32 = np.uint32


def _rol(x, r):
    r = _U32(r)
    return (x << r) | (x >> _U32(32 - r))


def _threefry2x32(k1, k2, x1, x2):
    ks0 = _U32(k1) * np.ones_like(x1)
    ks1 = _U32(k2) * np.ones_like(x1)
    ks2 = ks0 ^ ks1 ^ _U32(0x1BD11BDA)
    rot1 = (13, 15, 26, 6)
    rot2 = (17, 29, 16, 24)
    a = x1 + ks0
    b = x2 + ks1

    def rounds(a, b, rots):
        for r in rots:
            a = a + b
            b = _rol(b, r)
            b = a ^ b
        return a, b

    a, b = rounds(a, b, rot1); a = a + ks1; b = b + ks2 + _U32(1)
    a, b = rounds(a, b, rot2); a = a + ks2; b = b + ks0 + _U32(2)
    a, b = rounds(a, b, rot1); a = a + ks0; b = b + ks1 + _U32(3)
    a, b = rounds(a, b, rot2); a = a + ks1; b = b + ks2 + _U32(4)
    a, b = rounds(a, b, rot1); a = a + ks2; b = b + ks0 + _U32(5)
    return a, b


def _uniform_01(key, n):
    a, b = _threefry2x32(key[0], key[1], np.zeros(n, _U32),
                         np.arange(n, dtype=_U32))
    bits = a ^ b
    fb = (bits >> _U32(9)) | _U32(0x3F800000)
    return fb.view(np.float32) - np.float32(1.0)


def _sampling_perms():
    pp = np.empty((C, B), np.int32)
    pn = np.empty((C, B), np.int32)
    for c in range(C):
        a, b = _threefry2x32(0, 42, np.array([0], _U32), np.array([c], _U32))
        kc = (a[0], b[0])
        a, b = _threefry2x32(kc[0], kc[1], np.array([0, 0], _U32),
                             np.array([0, 1], _U32))
        kp, kn = (a[0], b[0]), (a[1], b[1])
        pp[c] = np.argsort(_uniform_01(kp, B), kind="stable")
        pn[c] = np.argsort(_uniform_01(kn, B), kind="stable")
    return (pp.reshape(C, B // LANES, LANES),
            pn.reshape(C, B // LANES, LANES))


_PP, _PN = _sampling_perms()


def _chunk_scan(trow, tv, idx, cj, want, invert, out_idx):
    t = plsc.load_gather(trow, [idx])
    m = (t != tv) if invert else (t == tv)
    mi = m.astype(jnp.int32)
    cs = plsc.cumsum(mi)
    dst = cj + cs - 1
    keep = jnp.logical_and(m, dst < want)
    dst_safe = jnp.where(keep, dst, 0)
    plsc.store_scatter(out_idx, [dst_safe], idx, mask=keep)
    return cj + jnp.sum(mi)


def _cont_scan(perm_hbm, c, trow, tv, out_idx, want, invert, start_cnt,
               start_sub, cwin, sem):
    def sub_work(w, cnt):
        pltpu.async_copy(perm_hbm.at[c, pl.ds(w * SUB_CHUNKS, SUB_CHUNKS)],
                         cwin, sem).wait()
        for j in range(SUB_CHUNKS):
            cnt = _chunk_scan(trow, tv, cwin[j], cnt, want, invert,
                              out_idx)
        return cnt

    def sub_body(w, cnt):
        return lax.cond(cnt < want, lambda x: sub_work(w, x), lambda x: x,
                        cnt)

    return lax.fori_loop(start_sub, NUM_SUBS, sub_body, start_cnt)


_KCLASSES = (C + NUM_WORKERS - 1) // NUM_WORKERS


def _sc_body(lT, tT, pp, pn, pos_out, neg_out,
             pwin, nwin, trowA, trowB, lrowA, lrowB, cwin,
             pidx, nidx, pvals, nvals,
             semw, semt, semc, semo):
    wid = lax.axis_index("s") * NUM_CORES + lax.axis_index("c")

    def cmk(k):
        return jnp.minimum(wid + NUM_WORKERS * k, C - 1)

    stage = []
    for k in range(_KCLASSES):
        cm = cmk(k)
        stage.append([
            pltpu.async_copy(pp.at[cm, pl.ds(0, FAST_CHUNKS)],
                             pwin.at[pl.ds(k * FAST_CHUNKS, FAST_CHUNKS)],
                             semw),
            pltpu.async_copy(pn.at[cm, pl.ds(0, FAST_CHUNKS)],
                             nwin.at[pl.ds(k * FAST_CHUNKS, FAST_CHUNKS)],
                             semw),
        ])
    tbufs = [trowA, trowB]
    lbufs = [lrowA, lrowB]
    trow_h = pltpu.async_copy(tT.at[cmk(0)], trowA, semt)
    lrow_h = pltpu.async_copy(lT.at[cmk(0)], lrowA, semt)

    def do_class(k, c, trow, lrow):
        def fast_scan(win, tv, out_idx, want):
            def chunk_body(j, cj):
                def work(x):
                    return _chunk_scan(trow, tv, win[k * FAST_CHUNKS + j],
                                       x, want, False, out_idx)
                return lax.cond(cj < want, work, lambda x: x, cj)
            return lax.fori_loop(0, FAST_CHUNKS, chunk_body, jnp.int32(0))

        npos = fast_scan(pwin, 1, pidx, MAX_POS)
        nneg = fast_scan(nwin, 0, nidx, MAX_NEG)

        npos = lax.cond(
            npos < MAX_POS,
            lambda x: _cont_scan(pp, c, trow, 1, pidx, MAX_POS, False,
                                 x, FAST // SUB, cwin, semc),
            lambda x: x, npos)
        nneg = lax.cond(
            nneg < MAX_NEG,
            lambda x: _cont_scan(pn, c, trow, 0, nidx, MAX_NEG, False,
                                 x, FAST // SUB, cwin, semc),
            lambda x: x, nneg)

        @pl.when(npos < MAX_POS)
        def _():
            _cont_scan(pp, c, trow, 1, pidx, MAX_POS, True, npos, 0,
                       cwin, semc)

        @pl.when(nneg < MAX_NEG)
        def _():
            _cont_scan(pn, c, trow, 0, nidx, MAX_NEG, True, nneg, 0,
                       cwin, semc)

        for s in range(MAX_POS // LANES):
            sel = pidx[pl.ds(s * LANES, LANES)]
            pvals[pl.ds(s * LANES, LANES)] = plsc.load_gather(lrow, [sel])
        for s in range(MAX_NEG // LANES):
            sel = nidx[pl.ds(s * LANES, LANES)]
            nvals[pl.ds(s * LANES, LANES)] = plsc.load_gather(lrow, [sel])
        h1 = pltpu.async_copy(pvals, pos_out.at[c], semo)
        h2 = pltpu.async_copy(nvals, neg_out.at[c], semo)
        h1.wait()
        h2.wait()

    for k in range(_KCLASSES):
        c = wid + NUM_WORKERS * k
        for h in stage[k]:
            h.wait()
        trow_h.wait()
        lrow_h.wait()
        if k + 1 < _KCLASSES:
            trow_h = pltpu.async_copy(tT.at[cmk(k + 1)],
                                      tbufs[(k + 1) % 2], semt)
            lrow_h = pltpu.async_copy(lT.at[cmk(k + 1)],
                                      lbufs[(k + 1) % 2], semt)

        @pl.when(c < C)
        def _(k=k, c=c):
            do_class(k, c, tbufs[k % 2], lbufs[k % 2])


def _make_sc_sampler():
    mesh = plsc.VectorSubcoreMesh(core_axis_name="c", subcore_axis_name="s",
                                  num_cores=NUM_CORES,
                                  num_subcores=NUM_SUBCORES)
    return pl.kernel(
        _sc_body,
        out_type=[
            jax.ShapeDtypeStruct((C, MAX_POS), jnp.float32),
            jax.ShapeDtypeStruct((C, MAX_NEG), jnp.float32),
        ],
        mesh=mesh,
        compiler_params=pltpu.CompilerParams(needs_layout_passes=False,
                                             use_tc_tiling_on_sc=False),
        scratch_types=[
            pltpu.VMEM((_KCLASSES * FAST_CHUNKS, LANES), jnp.int32),
            pltpu.VMEM((_KCLASSES * FAST_CHUNKS, LANES), jnp.int32),
            pltpu.VMEM((B,), jnp.int32),
            pltpu.VMEM((B,), jnp.int32),
            pltpu.VMEM((B,), jnp.float32),
            pltpu.VMEM((B,), jnp.float32),
            pltpu.VMEM((SUB_CHUNKS, LANES), jnp.int32),
            pltpu.VMEM((MAX_POS,), jnp.int32),
            pltpu.VMEM((MAX_NEG,), jnp.int32),
            pltpu.VMEM((MAX_POS,), jnp.float32),
            pltpu.VMEM((MAX_NEG,), jnp.float32),
            pltpu.SemaphoreType.DMA,
            pltpu.SemaphoreType.DMA,
            pltpu.SemaphoreType.DMA,
            pltpu.SemaphoreType.DMA,
        ],
    )


def _loss_body(p_ref, n_ref, o_ref):
    n = n_ref[...]
    total = jnp.float32(0.0)
    for i in range(MAX_POS):
        d = n - p_ref[:, i][:, None]
        total = total + jnp.sum(jnp.logaddexp(d, 0.0))
    o_ref[0, 0] = total / jnp.float32(C * MAX_POS * MAX_NEG)


def _tc_loss(pos_vals, neg_vals):
    return pl.pallas_call(
        _loss_body,
        out_shape=jax.ShapeDtypeStruct((1, 1), jnp.float32),
        out_specs=pl.BlockSpec(memory_space=pltpu.SMEM),
    )(pos_vals, neg_vals)


def kernel(logits, targets):
    lT = jnp.transpose(logits)
    tT = jnp.transpose(targets.astype(jnp.int32))
    sampler = _make_sc_sampler()
    pos_vals, neg_vals = sampler(lT, tT, _PP, _PN)
    loss = _tc_loss(pos_vals, neg_vals)
    return jnp.reshape(loss, ())


# speedup vs baseline: 1.4117x; 1.1893x over previous
import jax
import jax.numpy as jnp
import numpy as np
from jax import lax
from jax.experimental import pallas as pl
from jax.experimental.pallas import tpu as pltpu
from jax.experimental.pallas import tpu_sc as plsc

B = 16384
C = 100
MAX_POS = 32
MAX_NEG = 64

NUM_CORES = 2
NUM_SUBCORES = 16
NUM_WORKERS = NUM_CORES * NUM_SUBCORES
LANES = 16

FAST = 512
FAST_CHUNKS = FAST // LANES
SUB = 128
SUB_CHUNKS = SUB // LANES
NUM_SUBS = B // SUB
MWORDS = B // 32

_U32 = np.uint32


def _rol(x, r):
    r = _U32(r)
    return (x << r) | (x >> _U32(32 - r))


def _threefry2x32(k1, k2, x1, x2):
    ks0 = _U32(k1) * np.ones_like(x1)
    ks1 = _U32(k2) * np.ones_like(x1)
    ks2 = ks0 ^ ks1 ^ _U32(0x1BD11BDA)
    rot1 = (13, 15, 26, 6)
    rot2 = (17, 29, 16, 24)
    a = x1 + ks0
    b = x2 + ks1

    def rounds(a, b, rots):
        for r in rots:
            a = a + b
            b = _rol(b, r)
            b = a ^ b
        return a, b

    a, b = rounds(a, b, rot1); a = a + ks1; b = b + ks2 + _U32(1)
    a, b = rounds(a, b, rot2); a = a + ks2; b = b + ks0 + _U32(2)
    a, b = rounds(a, b, rot1); a = a + ks0; b = b + ks1 + _U32(3)
    a, b = rounds(a, b, rot2); a = a + ks1; b = b + ks2 + _U32(4)
    a, b = rounds(a, b, rot1); a = a + ks2; b = b + ks0 + _U32(5)
    return a, b


def _uniform_01(key, n):
    a, b = _threefry2x32(key[0], key[1], np.zeros(n, _U32),
                         np.arange(n, dtype=_U32))
    bits = a ^ b
    fb = (bits >> _U32(9)) | _U32(0x3F800000)
    return fb.view(np.float32) - np.float32(1.0)


def _sampling_perms():
    pp = np.empty((C, B), np.int32)
    pn = np.empty((C, B), np.int32)
    for c in range(C):
        a, b = _threefry2x32(0, 42, np.array([0], _U32), np.array([c], _U32))
        kc = (a[0], b[0])
        a, b = _threefry2x32(kc[0], kc[1], np.array([0, 0], _U32),
                             np.array([0, 1], _U32))
        kp, kn = (a[0], b[0]), (a[1], b[1])
        pp[c] = np.argsort(_uniform_01(kp, B), kind="stable")
        pn[c] = np.argsort(_uniform_01(kn, B), kind="stable")
    return (pp.reshape(C, B // LANES, LANES),
            pn.reshape(C, B // LANES, LANES))


_PP, _PN = _sampling_perms()


def _chunk_scan(trow, tv, idx, cj, want, invert, out_idx):
    t = plsc.load_gather(trow, [idx])
    m = (t != tv) if invert else (t == tv)
    mi = m.astype(jnp.int32)
    cs = plsc.cumsum(mi)
    dst = cj + cs - 1
    keep = jnp.logical_and(m, dst < want)
    dst_safe = jnp.where(keep, dst, 0)
    plsc.store_scatter(out_idx, [dst_safe], idx, mask=keep)
    return cj + jnp.sum(mi)


def _cont_scan(perm_hbm, c, trow, tv, out_idx, want, invert, start_cnt,
               start_sub, cwin, sem):
    def sub_work(w, cnt):
        pltpu.async_copy(perm_hbm.at[c, pl.ds(w * SUB_CHUNKS, SUB_CHUNKS)],
                         cwin, sem).wait()
        for j in range(SUB_CHUNKS):
            cnt = _chunk_scan(trow, tv, cwin[j], cnt, want, invert,
                              out_idx)
        return cnt

    def sub_body(w, cnt):
        return lax.cond(cnt < want, lambda x: sub_work(w, x), lambda x: x,
                        cnt)

    return lax.fori_loop(start_sub, NUM_SUBS, sub_body, start_cnt)


_KCLASSES = (C + NUM_WORKERS - 1) // NUM_WORKERS


def _sc_body(lT, tT, pp, pn, pos_out, neg_out,
             pwin, nwin, trowA, trowB, lrowA, lrowB, cwin,
             pidx, nidx, pvals, nvals,
             semw, semt, semc, semo):
    wid = lax.axis_index("s") * NUM_CORES + lax.axis_index("c")

    def cmk(k):
        return jnp.minimum(wid + NUM_WORKERS * k, C - 1)

    stage = []
    for k in range(_KCLASSES):
        cm = cmk(k)
        stage.append([
            pltpu.async_copy(pp.at[cm, pl.ds(0, FAST_CHUNKS)],
                             pwin.at[pl.ds(k * FAST_CHUNKS, FAST_CHUNKS)],
                             semw),
            pltpu.async_copy(pn.at[cm, pl.ds(0, FAST_CHUNKS)],
                             nwin.at[pl.ds(k * FAST_CHUNKS, FAST_CHUNKS)],
                             semw),
        ])
    tbufs = [trowA, trowB]
    lbufs = [lrowA, lrowB]
    trow_h = pltpu.async_copy(tT.at[cmk(0)], trowA, semt)
    lrow_h = pltpu.async_copy(lT.at[cmk(0)], lrowA, semt)

    def do_class(k, c, trow, lrow):
        def fast_scan(win, tv, out_idx, want):
            def chunk_body(j, cj):
                def work(x):
                    return _chunk_scan(trow, tv, win[k * FAST_CHUNKS + j],
                                       x, want, False, out_idx)
                return lax.cond(cj < want, work, lambda x: x, cj)
            return lax.fori_loop(0, FAST_CHUNKS, chunk_body, jnp.int32(0))

        npos = fast_scan(pwin, 1, pidx, MAX_POS)
        nneg = fast_scan(nwin, 0, nidx, MAX_NEG)

        npos = lax.cond(
            npos < MAX_POS,
            lambda x: _cont_scan(pp, c, trow, 1, pidx, MAX_POS, False,
                                 x, FAST // SUB, cwin, semc),
            lambda x: x, npos)
        nneg = lax.cond(
            nneg < MAX_NEG,
            lambda x: _cont_scan(pn, c, trow, 0, nidx, MAX_NEG, False,
                                 x, FAST // SUB, cwin, semc),
            lambda x: x, nneg)

        @pl.when(npos < MAX_POS)
        def _():
            _cont_scan(pp, c, trow, 1, pidx, MAX_POS, True, npos, 0,
                       cwin, semc)

        @pl.when(nneg < MAX_NEG)
        def _():
            _cont_scan(pn, c, trow, 0, nidx, MAX_NEG, True, nneg, 0,
                       cwin, semc)

        for s in range(MAX_POS // LANES):
            sel = pidx[pl.ds(s * LANES, LANES)]
            pvals[pl.ds(s * LANES, LANES)] = plsc.load_gather(lrow, [sel])
        for s in range(MAX_NEG // LANES):
            sel = nidx[pl.ds(s * LANES, LANES)]
            nvals[pl.ds(s * LANES, LANES)] = plsc.load_gather(lrow, [sel])
        h1 = pltpu.async_copy(pvals, pos_out.at[c], semo)
        h2 = pltpu.async_copy(nvals, neg_out.at[c], semo)
        h1.wait()
        h2.wait()

    for k in range(_KCLASSES):
        c = wid + NUM_WORKERS * k
        for h in stage[k]:
            h.wait()
        trow_h.wait()
        lrow_h.wait()
        if k + 1 < _KCLASSES:
            trow_h = pltpu.async_copy(tT.at[cmk(k + 1)],
                                      tbufs[(k + 1) % 2], semt)
            lrow_h = pltpu.async_copy(lT.at[cmk(k + 1)],
                                      lbufs[(k + 1) % 2], semt)

        @pl.when(c < C)
        def _(k=k, c=c):
            do_class(k, c, tbufs[k % 2], lbufs[k % 2])


def _make_sc_sampler():
    mesh = plsc.VectorSubcoreMesh(core_axis_name="c", subcore_axis_name="s",
                                  num_cores=NUM_CORES,
                                  num_subcores=NUM_SUBCORES)
    return pl.kernel(
        _sc_body,
        out_type=[
            jax.ShapeDtypeStruct((C, MAX_POS), jnp.float32),
            jax.ShapeDtypeStruct((C, MAX_NEG), jnp.float32),
        ],
        mesh=mesh,
        compiler_params=pltpu.CompilerParams(needs_layout_passes=False),
        scratch_types=[
            pltpu.VMEM((_KCLASSES * FAST_CHUNKS, LANES), jnp.int32),
            pltpu.VMEM((_KCLASSES * FAST_CHUNKS, LANES), jnp.int32),
            pltpu.VMEM((B,), jnp.int32),
            pltpu.VMEM((B,), jnp.int32),
            pltpu.VMEM((B,), jnp.float32),
            pltpu.VMEM((B,), jnp.float32),
            pltpu.VMEM((SUB_CHUNKS, LANES), jnp.int32),
            pltpu.VMEM((MAX_POS,), jnp.int32),
            pltpu.VMEM((MAX_NEG,), jnp.int32),
            pltpu.VMEM((MAX_POS,), jnp.float32),
            pltpu.VMEM((MAX_NEG,), jnp.float32),
            pltpu.SemaphoreType.DMA,
            pltpu.SemaphoreType.DMA,
            pltpu.SemaphoreType.DMA,
            pltpu.SemaphoreType.DMA,
        ],
    )


def _loss_body(p_ref, n_ref, o_ref):
    n = n_ref[...]
    total = jnp.float32(0.0)
    for i in range(MAX_POS):
        d = n - p_ref[:, i][:, None]
        total = total + jnp.sum(jnp.logaddexp(d, 0.0))
    o_ref[0, 0] = total / jnp.float32(C * MAX_POS * MAX_NEG)


def _tc_loss(pos_vals, neg_vals):
    return pl.pallas_call(
        _loss_body,
        out_shape=jax.ShapeDtypeStruct((1, 1), jnp.float32),
        out_specs=pl.BlockSpec(memory_space=pltpu.SMEM),
    )(pos_vals, neg_vals)


def kernel(logits, targets):
    lT = jnp.transpose(logits)
    tT = jnp.transpose(targets.astype(jnp.int32))
    sampler = _make_sc_sampler()
    pos_vals, neg_vals = sampler(lT, tT, _PP, _PN)
    loss = _tc_loss(pos_vals, neg_vals)
    return jnp.reshape(loss, ())
